# Initial kernel scaffold; baseline (speedup 1.0000x reference)
import jax
import jax.numpy as jnp
from jax.experimental import pallas as pl


def kernel(x, seg, tok_table, seg_table, gamma, beta):
    raise NotImplementedError("write your pallas kernel here")


# speedup vs baseline: 1.5015x; 1.5015x over previous
import functools

import jax
import jax.numpy as jnp
from jax import lax
from jax.experimental import pallas as pl
from jax.experimental.pallas import tpu as pltpu
from jax.experimental.pallas import tpu_sc as plsc

VOCAB = 9
NSEG = 2
NROWS = VOCAB * NSEG
D = 1024


def _prep_kernel(x_ref, seg_ref, tok_ref, segt_ref, gamma_ref, beta_ref,
                 idx_ref, f_ref):
    t = tok_ref[...]
    s0 = segt_ref[0:1, :]
    s1 = segt_ref[1:2, :]
    e = jnp.concatenate([t + s0, t + s1], axis=0)
    mean = jnp.mean(e, axis=-1, keepdims=True)
    ctr = e - mean
    var = jnp.mean(ctr * ctr, axis=-1, keepdims=True)
    normed = ctr * lax.rsqrt(var + 1e-5)
    f_ref[...] = normed * gamma_ref[...] + beta_ref[...]
    idx_ref[...] = x_ref[...] + VOCAB * seg_ref[...]


def _make_sc_gather(n_tokens):
    info = plsc.get_sparse_core_info()
    nc, ns = info.num_cores, info.num_subcores
    nw = nc * ns
    per_w = n_tokens // nw
    chunk = 32
    n_chunks = per_w // chunk

    mesh = plsc.VectorSubcoreMesh(core_axis_name="c", subcore_axis_name="s")

    @functools.partial(
        pl.kernel,
        mesh=mesh,
        out_type=jax.ShapeDtypeStruct((n_tokens, D), jnp.float32),
        scratch_types=[
            pltpu.VMEM((per_w,), jnp.int32),
            pltpu.VMEM((chunk, D), jnp.float32),
            pltpu.VMEM((chunk, D), jnp.float32),
            pltpu.SemaphoreType.DMA,
            pltpu.SemaphoreType.DMA,
        ],
    )
    def sc_gather(f_hbm, idx_hbm, out_hbm, idx_v, buf0, buf1, sem0, sem1):
        wid = lax.axis_index("s") * nc + lax.axis_index("c")
        base = wid * per_w
        pltpu.sync_copy(idx_hbm.at[pl.ds(base, per_w)], idx_v)
        bufs = (buf0, buf1)
        sems = (sem0, sem1)
        copies = [None, None]
        copies[0] = pltpu.async_copy(
            f_hbm.at[idx_v.at[pl.ds(0, chunk)]], bufs[0], sems[0])
        for c in range(n_chunks):
            nxt = (c + 1) % 2
            if c + 1 < n_chunks:
                copies[nxt] = pltpu.async_copy(
                    f_hbm.at[idx_v.at[pl.ds((c + 1) * chunk, chunk)]],
                    bufs[nxt], sems[nxt])
            copies[c % 2].wait()
            pltpu.sync_copy(bufs[c % 2],
                            out_hbm.at[pl.ds(base + c * chunk, chunk)])

    return sc_gather


def kernel(x, seg, tok_table, seg_table, gamma, beta):
    B, L = x.shape
    n_tokens = B * L

    idx2d, ftab = pl.pallas_call(
        _prep_kernel,
        out_shape=(
            jax.ShapeDtypeStruct((n_tokens // 128, 128), jnp.int32),
            jax.ShapeDtypeStruct((NROWS, D), jnp.float32),
        ),
    )(
        x.reshape(n_tokens // 128, 128),
        seg.reshape(n_tokens // 128, 128),
        tok_table,
        seg_table,
        gamma.reshape(1, D),
        beta.reshape(1, D),
    )

    idx = idx2d.reshape(n_tokens)
    out = _make_sc_gather(n_tokens)(ftab, idx)
    return out.reshape(B, L, D)


# speedup vs baseline: 4.0250x; 2.6806x over previous
import functools

import jax
import jax.numpy as jnp
from jax import lax
from jax.experimental import pallas as pl
from jax.experimental.pallas import tpu as pltpu
from jax.experimental.pallas import tpu_sc as plsc

VOCAB = 9
NSEG = 2
NROWS = VOCAB * NSEG
D = 1024


def _prep_kernel(nworkers, wdiv, x_ref, seg_ref, tok_ref, segt_ref,
                 gamma_ref, beta_ref, idx_ref, f_ref):
    t = tok_ref[...]
    s0 = segt_ref[0:1, :]
    s1 = segt_ref[1:2, :]
    e = jnp.concatenate([t + s0, t + s1], axis=0)
    mean = jnp.mean(e, axis=-1, keepdims=True)
    ctr = e - mean
    var = jnp.mean(ctr * ctr, axis=-1, keepdims=True)
    normed = ctr * lax.rsqrt(var + 1e-5)
    f = normed * gamma_ref[...] + beta_ref[...]
    f_ref[...] = jnp.broadcast_to(f[None], (nworkers, NROWS, D)).reshape(
        nworkers * NROWS, D)
    w = lax.broadcasted_iota(jnp.int32, x_ref.shape, 0) // wdiv
    idx_ref[...] = x_ref[...] + VOCAB * seg_ref[...] + NROWS * w


def _make_sc_gather(n_tokens):
    info = plsc.get_sparse_core_info()
    nc, ns = info.num_cores, info.num_subcores
    nw = nc * ns
    per_w = n_tokens // nw
    chunk = 32
    n_chunks = per_w // chunk

    mesh = plsc.VectorSubcoreMesh(core_axis_name="c", subcore_axis_name="s")

    @functools.partial(
        pl.kernel,
        mesh=mesh,
        out_type=jax.ShapeDtypeStruct((n_tokens, D), jnp.float32),
        scratch_types=[
            pltpu.VMEM((per_w,), jnp.int32),
            pltpu.VMEM((chunk, D), jnp.float32),
            pltpu.VMEM((chunk, D), jnp.float32),
            pltpu.SemaphoreType.DMA,
            pltpu.SemaphoreType.DMA,
        ],
    )
    def sc_gather(f_hbm, idx_hbm, out_hbm, idx_v, buf0, buf1, sem0, sem1):
        wid = lax.axis_index("s") * nc + lax.axis_index("c")
        base = wid * per_w
        pltpu.sync_copy(idx_hbm.at[pl.ds(base, per_w)], idx_v)
        bufs = (buf0, buf1)
        sems = (sem0, sem1)
        copies = [None, None]
        copies[0] = pltpu.async_copy(
            f_hbm.at[idx_v.at[pl.ds(0, chunk)]], bufs[0], sems[0])
        for c in range(n_chunks):
            nxt = (c + 1) % 2
            if c + 1 < n_chunks:
                copies[nxt] = pltpu.async_copy(
                    f_hbm.at[idx_v.at[pl.ds((c + 1) * chunk, chunk)]],
                    bufs[nxt], sems[nxt])
            copies[c % 2].wait()
            pltpu.sync_copy(bufs[c % 2],
                            out_hbm.at[pl.ds(base + c * chunk, chunk)])

    return sc_gather, nw, per_w


def kernel(x, seg, tok_table, seg_table, gamma, beta):
    B, L = x.shape
    n_tokens = B * L
    sc_gather, nw, per_w = _make_sc_gather(n_tokens)
    wdiv = per_w // 128

    idx2d, ftab = pl.pallas_call(
        functools.partial(_prep_kernel, nw, wdiv),
        out_shape=(
            jax.ShapeDtypeStruct((n_tokens // 128, 128), jnp.int32),
            jax.ShapeDtypeStruct((nw * NROWS, D), jnp.float32),
        ),
    )(
        x.reshape(n_tokens // 128, 128),
        seg.reshape(n_tokens // 128, 128),
        tok_table,
        seg_table,
        gamma.reshape(1, D),
        beta.reshape(1, D),
    )

    idx = idx2d.reshape(n_tokens)
    out = sc_gather(ftab, idx)
    return out.reshape(B, L, D)


# speedup vs baseline: 4.0861x; 1.0152x over previous
import functools

import jax
import jax.numpy as jnp
from jax import lax
from jax.experimental import pallas as pl
from jax.experimental.pallas import tpu as pltpu
from jax.experimental.pallas import tpu_sc as plsc

VOCAB = 9
NSEG = 2
NROWS = VOCAB * NSEG
D = 1024


def _prep_kernel(nworkers, wdiv, x_ref, seg_ref, tok_ref, segt_ref,
                 gamma_ref, beta_ref, idx_ref, f_ref):
    t = tok_ref[...]
    s0 = segt_ref[0:1, :]
    s1 = segt_ref[1:2, :]
    e = jnp.concatenate([t + s0, t + s1], axis=0)
    mean = jnp.mean(e, axis=-1, keepdims=True)
    ctr = e - mean
    var = jnp.mean(ctr * ctr, axis=-1, keepdims=True)
    normed = ctr * lax.rsqrt(var + 1e-5)
    f = normed * gamma_ref[...] + beta_ref[...]
    f_ref[...] = jnp.broadcast_to(f[None], (nworkers, NROWS, D)).reshape(
        nworkers * NROWS, D)
    w = lax.broadcasted_iota(jnp.int32, x_ref.shape, 0) // wdiv
    idx_ref[...] = x_ref[...] + VOCAB * seg_ref[...] + NROWS * w


def _make_sc_gather(n_tokens):
    info = plsc.get_sparse_core_info()
    nc, ns = info.num_cores, info.num_subcores
    nw = nc * ns
    per_w = n_tokens // nw
    chunk = 32
    n_chunks = per_w // chunk

    mesh = plsc.VectorSubcoreMesh(core_axis_name="c", subcore_axis_name="s")

    @functools.partial(
        pl.kernel,
        mesh=mesh,
        out_type=jax.ShapeDtypeStruct((n_tokens, D), jnp.float32),
        scratch_types=[
            pltpu.VMEM((per_w,), jnp.int32),
            pltpu.VMEM((chunk, D), jnp.float32),
            pltpu.VMEM((chunk, D), jnp.float32),
            pltpu.VMEM((chunk, D), jnp.float32),
            pltpu.SemaphoreType.DMA,
            pltpu.SemaphoreType.DMA,
            pltpu.SemaphoreType.DMA,
            pltpu.SemaphoreType.DMA,
            pltpu.SemaphoreType.DMA,
            pltpu.SemaphoreType.DMA,
        ],
    )
    def sc_gather(f_hbm, idx_hbm, out_hbm, idx_v,
                  buf0, buf1, buf2, gs0, gs1, gs2, ws0, ws1, ws2):
        wid = lax.axis_index("s") * nc + lax.axis_index("c")
        base = wid * per_w
        pltpu.sync_copy(idx_hbm.at[pl.ds(base, per_w)], idx_v)
        bufs = (buf0, buf1, buf2)
        gsems = (gs0, gs1, gs2)
        wsems = (ws0, ws1, ws2)

        def gather(c):
            return pltpu.async_copy(
                f_hbm.at[idx_v.at[pl.ds(c * chunk, chunk)]],
                bufs[c % 3], gsems[c % 3])

        def write(c):
            return pltpu.async_copy(
                bufs[c % 3], out_hbm.at[pl.ds(base + c * chunk, chunk)],
                wsems[c % 3])

        gcopies = [None, None, None]
        wcopies = [None, None, None]
        gcopies[0] = gather(0)
        gcopies[1] = gather(1)
        for c in range(n_chunks):
            nxt = c + 2
            if nxt < n_chunks:
                if c >= 1:
                    wcopies[nxt % 3].wait()
                gcopies[nxt % 3] = gather(nxt)
            gcopies[c % 3].wait()
            wcopies[c % 3] = write(c)
        for c in range(max(0, n_chunks - 3), n_chunks):
            wcopies[c % 3].wait()

    return sc_gather, nw, per_w


def kernel(x, seg, tok_table, seg_table, gamma, beta):
    B, L = x.shape
    n_tokens = B * L
    sc_gather, nw, per_w = _make_sc_gather(n_tokens)
    wdiv = per_w // 128

    idx2d, ftab = pl.pallas_call(
        functools.partial(_prep_kernel, nw, wdiv),
        out_shape=(
            jax.ShapeDtypeStruct((n_tokens // 128, 128), jnp.int32),
            jax.ShapeDtypeStruct((nw * NROWS, D), jnp.float32),
        ),
    )(
        x.reshape(n_tokens // 128, 128),
        seg.reshape(n_tokens // 128, 128),
        tok_table,
        seg_table,
        gamma.reshape(1, D),
        beta.reshape(1, D),
    )

    idx = idx2d.reshape(n_tokens)
    out = sc_gather(ftab, idx)
    return out.reshape(B, L, D)
